# Initial kernel scaffold; baseline (speedup 1.0000x reference)
import jax
import jax.numpy as jnp
from jax.experimental import pallas as pl


def kernel(x, weight, W_out):
    raise NotImplementedError("write your pallas kernel here")


# speedup vs baseline: 4.4516x; 4.4516x over previous
import functools

import jax
import jax.numpy as jnp
from jax import lax
from jax.experimental import pallas as pl
from jax.experimental.pallas import tpu as pltpu
from jax.experimental.pallas import tpu_sc as plsc

VOCAB = 100000
EMB = 300
LANES = 16
NUM_CORES = 2
NUM_SUBCORES = 16
NW = NUM_CORES * NUM_SUBCORES
BATCH = 4096
HIST = 50
ROWS_PER_W = BATCH // NW
MM_BLK = 2000


def _mm_body(w_ref, wo_ref, p_ref):
    p_ref[...] = jnp.dot(w_ref[...], wo_ref[...],
                         preferred_element_type=jnp.float32)


def _project_table(weight, w_pad):
    return pl.pallas_call(
        _mm_body,
        grid=(VOCAB // MM_BLK,),
        in_specs=[
            pl.BlockSpec((MM_BLK, EMB), lambda i: (i, 0)),
            pl.BlockSpec((EMB, LANES), lambda i: (0, 0)),
        ],
        out_specs=pl.BlockSpec((MM_BLK, LANES), lambda i: (i, 0)),
        out_shape=jax.ShapeDtypeStruct((VOCAB, LANES), jnp.float32),
    )(weight, w_pad)


def _sc_body(xt_hbm, p_hbm, out_hbm, idx_v, rows_v, out_v, sem):
    wid = lax.axis_index("s") * NUM_CORES + lax.axis_index("c")

    pltpu.sync_copy(xt_hbm.at[wid], idx_v)

    def fire(j, c):
        pltpu.make_async_copy(p_hbm.at[idx_v.at[j]], rows_v.at[j], sem).start()
        return c

    lax.fori_loop(0, HIST, fire, 0)

    def drain(j, c):
        pltpu.make_async_copy(p_hbm.at[idx_v.at[j]], rows_v.at[j], sem).wait()
        return c

    lax.fori_loop(0, HIST, drain, 0)

    def row(b, c):
        acc = rows_v[0, b]
        for j in range(1, HIST):
            acc = acc + rows_v[j, b]
        out_v[b] = acc
        return c

    lax.fori_loop(0, ROWS_PER_W, row, 0)

    pltpu.sync_copy(out_v, out_hbm.at[pl.ds(wid * ROWS_PER_W, ROWS_PER_W)])


_gather_pool = functools.partial(
    pl.kernel,
    mesh=plsc.VectorSubcoreMesh(core_axis_name="c", subcore_axis_name="s"),
    out_type=jax.ShapeDtypeStruct((BATCH, LANES), jnp.float32),
    scratch_types=[
        pltpu.VMEM((HIST, ROWS_PER_W), jnp.int32),
        pltpu.VMEM((HIST, ROWS_PER_W, LANES), jnp.float32),
        pltpu.VMEM((ROWS_PER_W, LANES), jnp.float32),
        pltpu.SemaphoreType.DMA,
    ],
    compiler_params=pltpu.CompilerParams(use_tc_tiling_on_sc=False),
)(_sc_body)


def kernel(x, weight, W_out):
    n_out = W_out.shape[0]
    w_pad = jnp.zeros((EMB, LANES), jnp.float32)
    w_pad = w_pad.at[:, :n_out].set(W_out.T.astype(jnp.float32) * (1.0 / HIST))
    p = _project_table(weight, w_pad)
    xt = x.astype(jnp.int32).reshape(NW, ROWS_PER_W, HIST).transpose(0, 2, 1)
    out16 = _gather_pool(xt, p)
    return out16[:, :n_out]


# speedup vs baseline: 4.6639x; 1.0477x over previous
import functools

import jax
import jax.numpy as jnp
from jax import lax
from jax.experimental import pallas as pl
from jax.experimental.pallas import tpu as pltpu
from jax.experimental.pallas import tpu_sc as plsc

VOCAB = 100000
EMB = 300
LANES = 16
NUM_CORES = 2
NUM_SUBCORES = 16
NW = NUM_CORES * NUM_SUBCORES
BATCH = 4096
HIST = 50
ROWS_PER_W = BATCH // NW
MM_BLK = 10000


def _mm_body(w_ref, wo_ref, p_ref):
    p_ref[...] = jnp.dot(w_ref[...], wo_ref[...],
                         preferred_element_type=jnp.float32)


def _project_table(weight, w_pad):
    return pl.pallas_call(
        _mm_body,
        grid=(VOCAB // MM_BLK,),
        in_specs=[
            pl.BlockSpec((MM_BLK, EMB), lambda i: (i, 0)),
            pl.BlockSpec((EMB, LANES), lambda i: (0, 0)),
        ],
        out_specs=pl.BlockSpec((MM_BLK, LANES), lambda i: (i, 0)),
        out_shape=jax.ShapeDtypeStruct((VOCAB, LANES), jnp.float32),
    )(weight, w_pad)


def _sc_body(xt_hbm, p_hbm, out_hbm, idx_v, rows_v, out_v, sem):
    wid = lax.axis_index("s") * NUM_CORES + lax.axis_index("c")

    pltpu.sync_copy(xt_hbm.at[wid], idx_v)

    def fire(j, c):
        pltpu.make_async_copy(p_hbm.at[idx_v.at[j]], rows_v.at[j], sem).start()
        return c

    lax.fori_loop(0, HIST, fire, 0)

    def drain(j, c):
        pltpu.make_async_copy(p_hbm.at[idx_v.at[j]], rows_v.at[j], sem).wait()
        return c

    lax.fori_loop(0, HIST, drain, 0)

    def row(b, c):
        acc = rows_v[0, b]
        for j in range(1, HIST):
            acc = acc + rows_v[j, b]
        out_v[b] = acc
        return c

    lax.fori_loop(0, ROWS_PER_W, row, 0)

    pltpu.sync_copy(out_v, out_hbm.at[pl.ds(wid * ROWS_PER_W, ROWS_PER_W)])


_gather_pool = functools.partial(
    pl.kernel,
    mesh=plsc.VectorSubcoreMesh(core_axis_name="c", subcore_axis_name="s"),
    out_type=jax.ShapeDtypeStruct((BATCH, LANES), jnp.float32),
    scratch_types=[
        pltpu.VMEM((HIST, ROWS_PER_W), jnp.int32),
        pltpu.VMEM((HIST, ROWS_PER_W, LANES), jnp.float32),
        pltpu.VMEM((ROWS_PER_W, LANES), jnp.float32),
        pltpu.SemaphoreType.DMA,
    ],
    compiler_params=pltpu.CompilerParams(use_tc_tiling_on_sc=False),
)(_sc_body)


def kernel(x, weight, W_out):
    n_out = W_out.shape[0]
    w_pad = jnp.zeros((EMB, LANES), jnp.float32)
    w_pad = w_pad.at[:, :n_out].set(W_out.T.astype(jnp.float32) * (1.0 / HIST))
    p = _project_table(weight, w_pad)
    xt = x.astype(jnp.int32).reshape(NW, ROWS_PER_W, HIST).transpose(0, 2, 1)
    out16 = _gather_pool(xt, p)
    return out16[:, :n_out]


# speedup vs baseline: 4.6782x; 1.0031x over previous
import functools

import jax
import jax.numpy as jnp
from jax import lax
from jax.experimental import pallas as pl
from jax.experimental.pallas import tpu as pltpu
from jax.experimental.pallas import tpu_sc as plsc

VOCAB = 100000
EMB = 300
LANES = 16
NUM_CORES = 2
NUM_SUBCORES = 16
NW = NUM_CORES * NUM_SUBCORES
BATCH = 4096
HIST = 50
ROWS_PER_W = BATCH // NW
MM_CHUNK = 2000
MM_NCHUNK = VOCAB // MM_CHUNK
MM_NBUF = 6


def _mm_body(w_hbm, wo_ref, p_ref, in_buf, sems):
    i = pl.program_id(0)

    @pl.when(i == 0)
    def _prime():
        for k in range(MM_NBUF):
            pltpu.make_async_copy(w_hbm.at[pl.ds(k * MM_CHUNK, MM_CHUNK)],
                                  in_buf.at[k], sems.at[k]).start()

    b = lax.rem(i, MM_NBUF)
    pltpu.make_async_copy(w_hbm.at[pl.ds(i * MM_CHUNK, MM_CHUNK)],
                          in_buf.at[b], sems.at[b]).wait()
    p_ref[...] = jnp.dot(in_buf[b], wo_ref[...],
                         preferred_element_type=jnp.float32)
    nxt = i + MM_NBUF

    @pl.when(nxt < MM_NCHUNK)
    def _refill():
        bn = lax.rem(nxt, MM_NBUF)
        pltpu.make_async_copy(w_hbm.at[pl.ds(nxt * MM_CHUNK, MM_CHUNK)],
                              in_buf.at[bn], sems.at[bn]).start()


def _project_table(weight, w_pad):
    return pl.pallas_call(
        _mm_body,
        grid=(MM_NCHUNK,),
        in_specs=[
            pl.BlockSpec(memory_space=pl.ANY),
            pl.BlockSpec((EMB, LANES), lambda i: (0, 0)),
        ],
        out_specs=pl.BlockSpec((MM_CHUNK, LANES), lambda i: (i, 0)),
        out_shape=jax.ShapeDtypeStruct((VOCAB, LANES), jnp.float32),
        scratch_shapes=[
            pltpu.VMEM((MM_NBUF, MM_CHUNK, EMB), jnp.float32),
            pltpu.SemaphoreType.DMA((MM_NBUF,)),
        ],
    )(weight, w_pad)


def _sc_body(xt_hbm, p_hbm, out_hbm, idx_v, rows_v, out_v, sem):
    wid = lax.axis_index("s") * NUM_CORES + lax.axis_index("c")

    pltpu.sync_copy(xt_hbm.at[wid], idx_v)

    def fire(j, c):
        pltpu.make_async_copy(p_hbm.at[idx_v.at[j]], rows_v.at[j], sem).start()
        return c

    lax.fori_loop(0, HIST, fire, 0)

    def drain(j, c):
        pltpu.make_async_copy(p_hbm.at[idx_v.at[j]], rows_v.at[j], sem).wait()
        return c

    lax.fori_loop(0, HIST, drain, 0)

    def row(b, c):
        acc = rows_v[0, b]
        for j in range(1, HIST):
            acc = acc + rows_v[j, b]
        out_v[b] = acc
        return c

    lax.fori_loop(0, ROWS_PER_W, row, 0)

    pltpu.sync_copy(out_v, out_hbm.at[pl.ds(wid * ROWS_PER_W, ROWS_PER_W)])


_gather_pool = functools.partial(
    pl.kernel,
    mesh=plsc.VectorSubcoreMesh(core_axis_name="c", subcore_axis_name="s"),
    out_type=jax.ShapeDtypeStruct((BATCH, LANES), jnp.float32),
    scratch_types=[
        pltpu.VMEM((HIST, ROWS_PER_W), jnp.int32),
        pltpu.VMEM((HIST, ROWS_PER_W, LANES), jnp.float32),
        pltpu.VMEM((ROWS_PER_W, LANES), jnp.float32),
        pltpu.SemaphoreType.DMA,
    ],
    compiler_params=pltpu.CompilerParams(use_tc_tiling_on_sc=False),
)(_sc_body)


def kernel(x, weight, W_out):
    n_out = W_out.shape[0]
    w_pad = jnp.zeros((EMB, LANES), jnp.float32)
    w_pad = w_pad.at[:, :n_out].set(W_out.T.astype(jnp.float32) * (1.0 / HIST))
    p = _project_table(weight, w_pad)
    xt = x.astype(jnp.int32).reshape(NW, ROWS_PER_W, HIST).transpose(0, 2, 1)
    out16 = _gather_pool(xt, p)
    return out16[:, :n_out]
